# baseline (speedup 1.0000x reference)
import functools

import jax
import jax.numpy as jnp
from jax import lax
from jax.experimental import pallas as pl
from jax.experimental.pallas import tpu as pltpu
from jax.experimental.pallas import tpu_sc as plsc

N = 10000
E = 40000
D = 64
DP = 128
D_EDGE = 16
D_EH = 128
STEPS = 6

NW = 32
E_W = 1280
E_PAD = NW * E_W
K_CH = E_W // 128
E_H = E_W // 2
K_H = K_CH // 2
N_ACC = 5120
ZROWS = N_ACC // 16
T_E = 512


def _lin0_body(x_ref, w_ref, b_ref, o_ref):
    r = jnp.maximum(
        jnp.dot(x_ref[...], w_ref[...], preferred_element_type=jnp.float32)
        + b_ref[...], 0.0)
    o_ref[...] = jnp.concatenate([r, jnp.zeros((N, DP - D), jnp.float32)], axis=1)


@jax.jit
def _lin0_call(n_feat, lin0_w, lin0_b):
    return pl.pallas_call(
        _lin0_body,
        out_shape=jax.ShapeDtypeStruct((N, DP), jnp.float32),
    )(n_feat, lin0_w, lin0_b.reshape(1, D))


def _msgs_body(x_ref, ef_ref, par_ref, w1_ref, b1_ref, w2_ref, b2_ref, o_ref):
    h = jnp.maximum(
        jnp.dot(ef_ref[...], w1_ref[...], preferred_element_type=jnp.float32)
        + b1_ref[...], 0.0)
    w = jnp.dot(h, w2_ref[...], preferred_element_type=jnp.float32) + b2_ref[...]
    x = x_ref[...]
    acc = jnp.zeros((T_E, 2 * D), jnp.float32)
    for j in range(D // 2):
        xa = jnp.broadcast_to(x[:, 2 * j:2 * j + 1], (T_E, D))
        xb = jnp.broadcast_to(x[:, 2 * j + 1:2 * j + 2], (T_E, D))
        xx = jnp.concatenate([xa, xb], axis=1)
        acc = acc + w[:, 2 * D * j:2 * D * (j + 1)] * xx
    msgs = acc[:, :D] + acc[:, D:]
    o_ref[...] = jnp.concatenate(
        [msgs * par_ref[:, 0:1], msgs * par_ref[:, 1:2]], axis=1)


@jax.jit
def _msgs_call(x_src, ef_p, par_p, en1_w, en1_b, en2_w, en2_b):
    return pl.pallas_call(
        _msgs_body,
        grid=(E_PAD // T_E,),
        in_specs=[
            pl.BlockSpec((T_E, DP), lambda i: (i, 0)),
            pl.BlockSpec((T_E, D_EDGE), lambda i: (i, 0)),
            pl.BlockSpec((T_E, 2), lambda i: (i, 0)),
            pl.BlockSpec((D_EDGE, D_EH), lambda i: (0, 0)),
            pl.BlockSpec((1, D_EH), lambda i: (0, 0)),
            pl.BlockSpec((D_EH, D * D), lambda i: (0, 0)),
            pl.BlockSpec((1, D * D), lambda i: (0, 0)),
        ],
        out_specs=pl.BlockSpec((T_E, DP), lambda i: (i, 0)),
        out_shape=jax.ShapeDtypeStruct((E_PAD, DP), jnp.float32),
    )(x_src, ef_p, par_p, en1_w, en1_b.reshape(1, D_EH), en2_w,
      en2_b.reshape(1, D * D))


def _gather_body(table_hbm, src_hbm, out_hbm, idx_v, rows_v, sem):
    wid = lax.axis_index("s") * 2 + lax.axis_index("c")
    pltpu.sync_copy(src_hbm.at[wid], idx_v)
    for h in range(2):
        cps = [
            pltpu.async_copy(table_hbm.at[idx_v.at[h * K_H + j]],
                             rows_v.at[pl.ds(j * 128, 128)], sem)
            for j in range(K_H)
        ]
        for cp in cps:
            cp.wait()
        pltpu.sync_copy(rows_v, out_hbm.at[pl.ds(wid * E_W + h * E_H, E_H)])


@jax.jit
def _gather_call(table, src_p):
    mesh = plsc.VectorSubcoreMesh(core_axis_name="c", subcore_axis_name="s")
    return pl.kernel(
        _gather_body,
        out_type=jax.ShapeDtypeStruct((E_PAD, DP), jnp.float32),
        mesh=mesh,
        scratch_types=[
            pltpu.VMEM((K_CH, 128), jnp.int32),
            pltpu.VMEM((E_H, DP), jnp.float32),
            pltpu.SemaphoreType.DMA,
        ],
    )(table, src_p)


def _scatter_body(msgs_hbm, dst_hbm, z_hbm, out_hbm, idx_v, rows_v, acc_sh, sem):
    c = lax.axis_index("c")
    s = lax.axis_index("s")
    wid = s * 2 + c
    pltpu.sync_copy(z_hbm, rows_v.at[pl.ds(0, ZROWS)])
    pltpu.sync_copy(rows_v.at[pl.ds(0, ZROWS)], acc_sh.at[pl.ds(s * ZROWS, ZROWS)])
    pltpu.sync_copy(dst_hbm.at[wid], idx_v)
    plsc.subcore_barrier()
    for h in range(2):
        pltpu.sync_copy(msgs_hbm.at[pl.ds(wid * E_W + h * E_H, E_H)], rows_v)
        for j in range(K_H):
            pltpu.sync_copy(rows_v.at[pl.ds(j * 128, 128)],
                            acc_sh.at[idx_v.at[h * K_H + j]], add=True)
    plsc.subcore_barrier()
    pltpu.sync_copy(acc_sh.at[pl.ds(s * ZROWS, ZROWS)], rows_v.at[pl.ds(0, ZROWS)])
    pltpu.sync_copy(rows_v.at[pl.ds(0, ZROWS)],
                    out_hbm.at[pl.ds(c * N_ACC + s * ZROWS, ZROWS)])


@jax.jit
def _scatter_call(msgs, dst_p, zblk):
    mesh = plsc.VectorSubcoreMesh(core_axis_name="c", subcore_axis_name="s")
    return pl.kernel(
        _scatter_body,
        out_type=jax.ShapeDtypeStruct((2 * N_ACC, DP), jnp.float32),
        mesh=mesh,
        scratch_types=[
            pltpu.VMEM((K_CH, 128), jnp.int32),
            pltpu.VMEM((E_H, DP), jnp.float32),
            pltpu.VMEM_SHARED((N_ACC, DP), jnp.float32),
            pltpu.SemaphoreType.DMA,
        ],
    )(msgs, dst_p, zblk)


def _update_body(p0_ref, p1_ref, out_ref, init_ref, mw1_ref, mw2_ref, mb_ref,
                 cb_ref, o_ref, *, add_init):
    out = out_ref[:, :D]
    neigh = p0_ref[:N, :] + p1_ref[:N, :]
    m = jnp.maximum(neigh + out + cb_ref[...], 0.0)
    r = (jnp.dot(m, mw1_ref[...], preferred_element_type=jnp.float32)
         + jnp.dot(out, mw2_ref[...], preferred_element_type=jnp.float32)
         + mb_ref[...])
    if add_init:
        r = r + init_ref[...]
        o_ref[...] = r
    else:
        o_ref[...] = jnp.concatenate(
            [r, jnp.zeros((N, DP - D), jnp.float32)], axis=1)


@functools.partial(jax.jit, static_argnames=("add_init",))
def _update_call(p0, p1, out, init, mw1, mw2, msg_b, conv_bias, add_init):
    return pl.pallas_call(
        functools.partial(_update_body, add_init=add_init),
        out_shape=jax.ShapeDtypeStruct((N, D if add_init else DP), jnp.float32),
    )(p0, p1, out, init, mw1, mw2, msg_b.reshape(1, D), conv_bias.reshape(1, D))


def kernel(n_feat, e_feat, lin0_w, lin0_b, en1_w, en1_b, en2_w, en2_b,
           msg_w, msg_b, conv_bias, edge_index):
    src = edge_index[0]
    dst = edge_index[1]
    src_p = jnp.zeros((E_PAD,), jnp.int32).at[:E].set(src).reshape(NW, K_CH, 128)
    dst_full = jnp.full((E_PAD,), N, jnp.int32).at[:E].set(dst)
    dst_p = (dst_full // 2).reshape(NW, K_CH, 128)
    parity = (dst_full % 2).astype(jnp.float32)
    par_p = jnp.stack([1.0 - parity, parity], axis=1)
    ef_p = jnp.zeros((E_PAD, D_EDGE), jnp.float32).at[:E].set(e_feat)
    zblk = jnp.zeros((ZROWS, DP), jnp.float32)
    mw1 = msg_w[:D]
    mw2 = msg_w[D:]

    out = _lin0_call(n_feat, lin0_w, lin0_b)
    for s in range(STEPS):
        x_src = _gather_call(out, src_p)
        msgs = _msgs_call(x_src, ef_p, par_p, en1_w, en1_b, en2_w, en2_b)
        parts = _scatter_call(msgs, dst_p, zblk)
        p0 = parts[:N_ACC].reshape(2 * N_ACC, D)
        p1 = parts[N_ACC:].reshape(2 * N_ACC, D)
        out = _update_call(p0, p1, out, n_feat, mw1, mw2, msg_b, conv_bias,
                           add_init=(s == STEPS - 1))
    return out


# speedup vs baseline: 1.5750x; 1.5750x over previous
import functools

import jax
import jax.numpy as jnp
from jax import lax
from jax.experimental import pallas as pl
from jax.experimental.pallas import tpu as pltpu
from jax.experimental.pallas import tpu_sc as plsc

N = 10000
E = 40000
D = 64
DP = 128
D_EDGE = 16
D_EH = 128
STEPS = 6

NW = 32
E_W = 1280
E_PAD = NW * E_W
K_CH = E_W // 128
E_H = E_W // 2
K_H = K_CH // 2
N_ACC = 5120
ZROWS = N_ACC // 16
T_E = 512


def _lin0_body(x_ref, w_ref, b_ref, o_ref):
    r = jnp.maximum(
        jnp.dot(x_ref[...], w_ref[...], preferred_element_type=jnp.float32)
        + b_ref[...], 0.0)
    o_ref[...] = jnp.concatenate([r, jnp.zeros((N, DP - D), jnp.float32)], axis=1)


@jax.jit
def _lin0_call(n_feat, lin0_w, lin0_b):
    return pl.pallas_call(
        _lin0_body,
        out_shape=jax.ShapeDtypeStruct((N, DP), jnp.float32),
    )(n_feat, lin0_w, lin0_b.reshape(1, D))


def _hpre_body(ef_ref, w1_ref, b1_ref, o_ref):
    o_ref[...] = jnp.maximum(
        jnp.dot(ef_ref[...], w1_ref[...], preferred_element_type=jnp.float32)
        + b1_ref[...], 0.0)


@jax.jit
def _hpre_call(ef_p, en1_w, en1_b):
    return pl.pallas_call(
        _hpre_body,
        grid=(8,),
        in_specs=[
            pl.BlockSpec((E_PAD // 8, D_EDGE), lambda i: (i, 0)),
            pl.BlockSpec((D_EDGE, D_EH), lambda i: (0, 0)),
            pl.BlockSpec((1, D_EH), lambda i: (0, 0)),
        ],
        out_specs=pl.BlockSpec((E_PAD // 8, D_EH), lambda i: (i, 0)),
        out_shape=jax.ShapeDtypeStruct((E_PAD, D_EH), jnp.float32),
    )(ef_p, en1_w, en1_b.reshape(1, D_EH))


def _msgs_body(x_ref, h_ref, par_ref, rep_ref, w2_ref, b2_ref, o_ref):
    w = jnp.dot(h_ref[...], w2_ref[...], preferred_element_type=jnp.float32) \
        + b2_ref[...]
    xrep = jnp.dot(x_ref[:, :D], rep_ref[...],
                   preferred_element_type=jnp.float32)
    s = w * xrep
    width = D * D // 2
    while width >= D:
        s = s[:, :width] + s[:, width:2 * width]
        width //= 2
    o_ref[...] = jnp.concatenate(
        [s * par_ref[:, 0:1], s * par_ref[:, 1:2]], axis=1)


@jax.jit
def _msgs_call(x_src, h_p, par_p, rep, en2_w, en2_b):
    return pl.pallas_call(
        _msgs_body,
        grid=(E_PAD // T_E,),
        in_specs=[
            pl.BlockSpec((T_E, DP), lambda i: (i, 0)),
            pl.BlockSpec((T_E, D_EH), lambda i: (i, 0)),
            pl.BlockSpec((T_E, 2), lambda i: (i, 0)),
            pl.BlockSpec((D, D * D), lambda i: (0, 0)),
            pl.BlockSpec((D_EH, D * D), lambda i: (0, 0)),
            pl.BlockSpec((1, D * D), lambda i: (0, 0)),
        ],
        out_specs=pl.BlockSpec((T_E, DP), lambda i: (i, 0)),
        out_shape=jax.ShapeDtypeStruct((E_PAD, DP), jnp.float32),
    )(x_src, h_p, par_p, rep, en2_w, en2_b.reshape(1, D * D))


def _gather_body(table_hbm, src_hbm, out_hbm, idx_v, rows_v, sem):
    wid = lax.axis_index("s") * 2 + lax.axis_index("c")
    pltpu.sync_copy(src_hbm.at[wid], idx_v)
    for h in range(2):
        cps = [
            pltpu.async_copy(table_hbm.at[idx_v.at[h * K_H + j]],
                             rows_v.at[pl.ds(j * 128, 128)], sem)
            for j in range(K_H)
        ]
        for cp in cps:
            cp.wait()
        pltpu.sync_copy(rows_v, out_hbm.at[pl.ds(wid * E_W + h * E_H, E_H)])


@jax.jit
def _gather_call(table, src_p):
    mesh = plsc.VectorSubcoreMesh(core_axis_name="c", subcore_axis_name="s")
    return pl.kernel(
        _gather_body,
        out_type=jax.ShapeDtypeStruct((E_PAD, DP), jnp.float32),
        mesh=mesh,
        scratch_types=[
            pltpu.VMEM((K_CH, 128), jnp.int32),
            pltpu.VMEM((E_H, DP), jnp.float32),
            pltpu.SemaphoreType.DMA,
        ],
    )(table, src_p)


def _scatter_body(msgs_hbm, dst_hbm, z_hbm, out_hbm, idx_v, rows_v, acc_sh, sem):
    c = lax.axis_index("c")
    s = lax.axis_index("s")
    wid = s * 2 + c
    pltpu.sync_copy(z_hbm, rows_v.at[pl.ds(0, ZROWS)])
    pltpu.sync_copy(rows_v.at[pl.ds(0, ZROWS)], acc_sh.at[pl.ds(s * ZROWS, ZROWS)])
    pltpu.sync_copy(dst_hbm.at[wid], idx_v)
    plsc.subcore_barrier()
    for h in range(2):
        pltpu.sync_copy(msgs_hbm.at[pl.ds(wid * E_W + h * E_H, E_H)], rows_v)
        for j in range(K_H):
            pltpu.sync_copy(rows_v.at[pl.ds(j * 128, 128)],
                            acc_sh.at[idx_v.at[h * K_H + j]], add=True)
    plsc.subcore_barrier()
    pltpu.sync_copy(acc_sh.at[pl.ds(s * ZROWS, ZROWS)], rows_v.at[pl.ds(0, ZROWS)])
    pltpu.sync_copy(rows_v.at[pl.ds(0, ZROWS)],
                    out_hbm.at[pl.ds(c * N_ACC + s * ZROWS, ZROWS)])


@jax.jit
def _scatter_call(msgs, dst_p, zblk):
    mesh = plsc.VectorSubcoreMesh(core_axis_name="c", subcore_axis_name="s")
    return pl.kernel(
        _scatter_body,
        out_type=jax.ShapeDtypeStruct((2 * N_ACC, DP), jnp.float32),
        mesh=mesh,
        scratch_types=[
            pltpu.VMEM((K_CH, 128), jnp.int32),
            pltpu.VMEM((E_H, DP), jnp.float32),
            pltpu.VMEM_SHARED((N_ACC, DP), jnp.float32),
            pltpu.SemaphoreType.DMA,
        ],
    )(msgs, dst_p, zblk)


def _update_body(p0_ref, p1_ref, out_ref, init_ref, mw1_ref, mw2_ref, mb_ref,
                 cb_ref, o_ref, *, add_init):
    out = out_ref[:, :D]
    neigh = p0_ref[:N, :] + p1_ref[:N, :]
    m = jnp.maximum(neigh + out + cb_ref[...], 0.0)
    r = (jnp.dot(m, mw1_ref[...], preferred_element_type=jnp.float32)
         + jnp.dot(out, mw2_ref[...], preferred_element_type=jnp.float32)
         + mb_ref[...])
    if add_init:
        r = r + init_ref[...]
        o_ref[...] = r
    else:
        o_ref[...] = jnp.concatenate(
            [r, jnp.zeros((N, DP - D), jnp.float32)], axis=1)


@functools.partial(jax.jit, static_argnames=("add_init",))
def _update_call(p0, p1, out, init, mw1, mw2, msg_b, conv_bias, add_init):
    return pl.pallas_call(
        functools.partial(_update_body, add_init=add_init),
        out_shape=jax.ShapeDtypeStruct((N, D if add_init else DP), jnp.float32),
    )(p0, p1, out, init, mw1, mw2, msg_b.reshape(1, D), conv_bias.reshape(1, D))


def kernel(n_feat, e_feat, lin0_w, lin0_b, en1_w, en1_b, en2_w, en2_b,
           msg_w, msg_b, conv_bias, edge_index):
    src = edge_index[0]
    dst = edge_index[1]
    src_p = jnp.zeros((E_PAD,), jnp.int32).at[:E].set(src).reshape(NW, K_CH, 128)
    dst_full = jnp.full((E_PAD,), N, jnp.int32).at[:E].set(dst)
    dst_p = (dst_full // 2).reshape(NW, K_CH, 128)
    parity = (dst_full % 2).astype(jnp.float32)
    par_p = jnp.stack([1.0 - parity, parity], axis=1)
    ef_p = jnp.zeros((E_PAD, D_EDGE), jnp.float32).at[:E].set(e_feat)
    zblk = jnp.zeros((ZROWS, DP), jnp.float32)
    rep = (jnp.arange(D * D, dtype=jnp.int32)[None, :] // D
           == jnp.arange(D, dtype=jnp.int32)[:, None]).astype(jnp.float32)
    mw1 = msg_w[:D]
    mw2 = msg_w[D:]

    out = _lin0_call(n_feat, lin0_w, lin0_b)
    h_p = _hpre_call(ef_p, en1_w, en1_b)
    for s in range(STEPS):
        x_src = _gather_call(out, src_p)
        msgs = _msgs_call(x_src, h_p, par_p, rep, en2_w, en2_b)
        parts = _scatter_call(msgs, dst_p, zblk)
        p0 = parts[:N_ACC].reshape(2 * N_ACC, D)
        p1 = parts[N_ACC:].reshape(2 * N_ACC, D)
        out = _update_call(p0, p1, out, n_feat, mw1, mw2, msg_b, conv_bias,
                           add_init=(s == STEPS - 1))
    return out


# speedup vs baseline: 1.6209x; 1.0291x over previous
import functools

import jax
import jax.numpy as jnp
from jax import lax
from jax.experimental import pallas as pl
from jax.experimental.pallas import tpu as pltpu
from jax.experimental.pallas import tpu_sc as plsc

N = 10000
E = 40000
D = 64
DP = 128
D_EDGE = 16
D_EH = 128
STEPS = 6

NW = 32
E_W = 1280
E_PAD = NW * E_W
K_CH = E_W // 128
E_H = E_W // 2
K_H = K_CH // 2
N_ACC = 5120
ZROWS = N_ACC // 16
T_E = 1024


def _lin0_body(x_ref, w_ref, b_ref, o_ref):
    r = jnp.maximum(
        jnp.dot(x_ref[...], w_ref[...], preferred_element_type=jnp.float32)
        + b_ref[...], 0.0)
    o_ref[...] = jnp.concatenate([r, jnp.zeros((N, DP - D), jnp.float32)], axis=1)


@jax.jit
def _lin0_call(n_feat, lin0_w, lin0_b):
    return pl.pallas_call(
        _lin0_body,
        out_shape=jax.ShapeDtypeStruct((N, DP), jnp.float32),
    )(n_feat, lin0_w, lin0_b.reshape(1, D))


def _hpre_body(ef_ref, w1_ref, b1_ref, o_ref):
    o_ref[...] = jnp.maximum(
        jnp.dot(ef_ref[...], w1_ref[...], preferred_element_type=jnp.float32)
        + b1_ref[...], 0.0)


@jax.jit
def _hpre_call(ef_p, en1_w, en1_b):
    return pl.pallas_call(
        _hpre_body,
        grid=(8,),
        in_specs=[
            pl.BlockSpec((E_PAD // 8, D_EDGE), lambda i: (i, 0)),
            pl.BlockSpec((D_EDGE, D_EH), lambda i: (0, 0)),
            pl.BlockSpec((1, D_EH), lambda i: (0, 0)),
        ],
        out_specs=pl.BlockSpec((E_PAD // 8, D_EH), lambda i: (i, 0)),
        out_shape=jax.ShapeDtypeStruct((E_PAD, D_EH), jnp.float32),
    )(ef_p, en1_w, en1_b.reshape(1, D_EH))


def _msgs_body(x_ref, h_ref, par_ref, rep_ref, w2_ref, b2_ref, o_ref):
    w = jnp.dot(h_ref[...], w2_ref[...], preferred_element_type=jnp.float32) \
        + b2_ref[...]
    xrep = jnp.dot(x_ref[:, :D], rep_ref[...],
                   preferred_element_type=jnp.float32)
    s = w * xrep
    width = D * D // 2
    while width >= D:
        s = s[:, :width] + s[:, width:2 * width]
        width //= 2
    o_ref[...] = jnp.concatenate(
        [s * par_ref[:, 0:1], s * par_ref[:, 1:2]], axis=1)


@jax.jit
def _msgs_call(x_src, h_p, par_p, rep, en2_w, en2_b):
    return pl.pallas_call(
        _msgs_body,
        grid=(E_PAD // T_E,),
        in_specs=[
            pl.BlockSpec((T_E, DP), lambda i: (i, 0)),
            pl.BlockSpec((T_E, D_EH), lambda i: (i, 0)),
            pl.BlockSpec((T_E, 2), lambda i: (i, 0)),
            pl.BlockSpec((D, D * D), lambda i: (0, 0)),
            pl.BlockSpec((D_EH, D * D), lambda i: (0, 0)),
            pl.BlockSpec((1, D * D), lambda i: (0, 0)),
        ],
        out_specs=pl.BlockSpec((T_E, DP), lambda i: (i, 0)),
        out_shape=jax.ShapeDtypeStruct((E_PAD, DP), jnp.float32),
    )(x_src, h_p, par_p, rep, en2_w, en2_b.reshape(1, D * D))


def _gather_body(table_hbm, src_hbm, out_hbm, idx_v, rows_a, rows_b, gsem,
                 wsem):
    wid = lax.axis_index("s") * 2 + lax.axis_index("c")
    pltpu.sync_copy(src_hbm.at[wid], idx_v)
    bufs = (rows_a, rows_b)
    writes = []
    for q in range(K_CH // 2):
        b = bufs[q % 2]
        if q >= 2:
            writes[q - 2].wait()
        g0 = pltpu.async_copy(table_hbm.at[idx_v.at[2 * q]],
                              b.at[pl.ds(0, 128)], gsem)
        g1 = pltpu.async_copy(table_hbm.at[idx_v.at[2 * q + 1]],
                              b.at[pl.ds(128, 128)], gsem)
        g0.wait()
        g1.wait()
        writes.append(pltpu.async_copy(
            b, out_hbm.at[pl.ds(wid * E_W + q * 256, 256)], wsem))
    writes[-2].wait()
    writes[-1].wait()


@jax.jit
def _gather_call(table, src_p):
    mesh = plsc.VectorSubcoreMesh(core_axis_name="c", subcore_axis_name="s")
    return pl.kernel(
        _gather_body,
        out_type=jax.ShapeDtypeStruct((E_PAD, DP), jnp.float32),
        mesh=mesh,
        scratch_types=[
            pltpu.VMEM((K_CH, 128), jnp.int32),
            pltpu.VMEM((256, DP), jnp.float32),
            pltpu.VMEM((256, DP), jnp.float32),
            pltpu.SemaphoreType.DMA,
            pltpu.SemaphoreType.DMA,
        ],
    )(table, src_p)


def _scatter_body(msgs_hbm, dst_hbm, z_hbm, out_hbm, idx_v, rows_v, acc_sh, sem):
    c = lax.axis_index("c")
    s = lax.axis_index("s")
    wid = s * 2 + c
    pltpu.sync_copy(z_hbm, rows_v.at[pl.ds(0, ZROWS)])
    pltpu.sync_copy(rows_v.at[pl.ds(0, ZROWS)], acc_sh.at[pl.ds(s * ZROWS, ZROWS)])
    pltpu.sync_copy(dst_hbm.at[wid], idx_v)
    plsc.subcore_barrier()
    for h in range(2):
        pltpu.sync_copy(msgs_hbm.at[pl.ds(wid * E_W + h * E_H, E_H)], rows_v)
        for j in range(K_H):
            pltpu.sync_copy(rows_v.at[pl.ds(j * 128, 128)],
                            acc_sh.at[idx_v.at[h * K_H + j]], add=True)
    plsc.subcore_barrier()
    pltpu.sync_copy(acc_sh.at[pl.ds(s * ZROWS, ZROWS)], rows_v.at[pl.ds(0, ZROWS)])
    pltpu.sync_copy(rows_v.at[pl.ds(0, ZROWS)],
                    out_hbm.at[pl.ds(c * N_ACC + s * ZROWS, ZROWS)])


@jax.jit
def _scatter_call(msgs, dst_p, zblk):
    mesh = plsc.VectorSubcoreMesh(core_axis_name="c", subcore_axis_name="s")
    return pl.kernel(
        _scatter_body,
        out_type=jax.ShapeDtypeStruct((2 * N_ACC, DP), jnp.float32),
        mesh=mesh,
        scratch_types=[
            pltpu.VMEM((K_CH, 128), jnp.int32),
            pltpu.VMEM((E_H, DP), jnp.float32),
            pltpu.VMEM_SHARED((N_ACC, DP), jnp.float32),
            pltpu.SemaphoreType.DMA,
        ],
    )(msgs, dst_p, zblk)


def _update_body(p0_ref, p1_ref, out_ref, init_ref, mw1_ref, mw2_ref, mb_ref,
                 cb_ref, o_ref, *, add_init):
    out = out_ref[:, :D]
    neigh = p0_ref[:N, :] + p1_ref[:N, :]
    m = jnp.maximum(neigh + out + cb_ref[...], 0.0)
    r = (jnp.dot(m, mw1_ref[...], preferred_element_type=jnp.float32)
         + jnp.dot(out, mw2_ref[...], preferred_element_type=jnp.float32)
         + mb_ref[...])
    if add_init:
        r = r + init_ref[...]
        o_ref[...] = r
    else:
        o_ref[...] = jnp.concatenate(
            [r, jnp.zeros((N, DP - D), jnp.float32)], axis=1)


@functools.partial(jax.jit, static_argnames=("add_init",))
def _update_call(p0, p1, out, init, mw1, mw2, msg_b, conv_bias, add_init):
    return pl.pallas_call(
        functools.partial(_update_body, add_init=add_init),
        out_shape=jax.ShapeDtypeStruct((N, D if add_init else DP), jnp.float32),
    )(p0, p1, out, init, mw1, mw2, msg_b.reshape(1, D), conv_bias.reshape(1, D))


def kernel(n_feat, e_feat, lin0_w, lin0_b, en1_w, en1_b, en2_w, en2_b,
           msg_w, msg_b, conv_bias, edge_index):
    src = edge_index[0]
    dst = edge_index[1]
    src_p = jnp.zeros((E_PAD,), jnp.int32).at[:E].set(src).reshape(NW, K_CH, 128)
    dst_full = jnp.full((E_PAD,), N, jnp.int32).at[:E].set(dst)
    dst_p = (dst_full // 2).reshape(NW, K_CH, 128)
    parity = (dst_full % 2).astype(jnp.float32)
    par_p = jnp.stack([1.0 - parity, parity], axis=1)
    ef_p = jnp.zeros((E_PAD, D_EDGE), jnp.float32).at[:E].set(e_feat)
    zblk = jnp.zeros((ZROWS, DP), jnp.float32)
    rep = (jnp.arange(D * D, dtype=jnp.int32)[None, :] // D
           == jnp.arange(D, dtype=jnp.int32)[:, None]).astype(jnp.float32)
    mw1 = msg_w[:D]
    mw2 = msg_w[D:]

    out = _lin0_call(n_feat, lin0_w, lin0_b)
    h_p = _hpre_call(ef_p, en1_w, en1_b)
    for s in range(STEPS):
        x_src = _gather_call(out, src_p)
        msgs = _msgs_call(x_src, h_p, par_p, rep, en2_w, en2_b)
        parts = _scatter_call(msgs, dst_p, zblk)
        p0 = parts[:N_ACC].reshape(2 * N_ACC, D)
        p1 = parts[N_ACC:].reshape(2 * N_ACC, D)
        out = _update_call(p0, p1, out, n_feat, mw1, mw2, msg_b, conv_bias,
                           add_init=(s == STEPS - 1))
    return out


# speedup vs baseline: 1.6291x; 1.0051x over previous
import functools

import jax
import jax.numpy as jnp
from jax import lax
from jax.experimental import pallas as pl
from jax.experimental.pallas import tpu as pltpu
from jax.experimental.pallas import tpu_sc as plsc

N = 10000
E = 40000
D = 64
DP = 128
D_EDGE = 16
D_EH = 128
STEPS = 6

NW = 32
E_W = 1280
E_PAD = NW * E_W
K_CH = E_W // 128
E_H = E_W // 2
K_H = K_CH // 2
N_ACC = 5120
ZROWS = N_ACC // 16
T_E = 1024


def _lin0_body(x_ref, w_ref, b_ref, o_ref):
    r = jnp.maximum(
        jnp.dot(x_ref[...], w_ref[...], preferred_element_type=jnp.float32)
        + b_ref[...], 0.0)
    o_ref[...] = jnp.concatenate([r, jnp.zeros((N, DP - D), jnp.float32)], axis=1)


@jax.jit
def _lin0_call(n_feat, lin0_w, lin0_b):
    return pl.pallas_call(
        _lin0_body,
        out_shape=jax.ShapeDtypeStruct((N, DP), jnp.float32),
    )(n_feat, lin0_w, lin0_b.reshape(1, D))


def _hpre_body(ef_ref, w1_ref, b1_ref, o_ref):
    o_ref[...] = jnp.maximum(
        jnp.dot(ef_ref[...], w1_ref[...], preferred_element_type=jnp.float32)
        + b1_ref[...], 0.0)


@jax.jit
def _hpre_call(ef_p, en1_w, en1_b):
    return pl.pallas_call(
        _hpre_body,
        grid=(8,),
        in_specs=[
            pl.BlockSpec((E_PAD // 8, D_EDGE), lambda i: (i, 0)),
            pl.BlockSpec((D_EDGE, D_EH), lambda i: (0, 0)),
            pl.BlockSpec((1, D_EH), lambda i: (0, 0)),
        ],
        out_specs=pl.BlockSpec((E_PAD // 8, D_EH), lambda i: (i, 0)),
        out_shape=jax.ShapeDtypeStruct((E_PAD, D_EH), jnp.float32),
    )(ef_p, en1_w, en1_b.reshape(1, D_EH))


def _msgs_body(x_ref, h_ref, par_ref, rep_ref, w2_ref, b2_ref, o_ref):
    w = jnp.dot(h_ref[...], w2_ref[...], preferred_element_type=jnp.float32) \
        + b2_ref[...]
    xrep = jnp.dot(x_ref[:, :D], rep_ref[...],
                   preferred_element_type=jnp.float32)
    s = w * xrep
    width = D * D // 2
    while width >= D:
        s = s[:, :width] + s[:, width:2 * width]
        width //= 2
    o_ref[...] = jnp.concatenate(
        [s * par_ref[:, 0:1], s * par_ref[:, 1:2]], axis=1)


@jax.jit
def _msgs_call(x_src, h_p, par_p, rep, en2_w, en2_b):
    return pl.pallas_call(
        _msgs_body,
        grid=(E_PAD // T_E,),
        in_specs=[
            pl.BlockSpec((T_E, DP), lambda i: (i, 0)),
            pl.BlockSpec((T_E, D_EH), lambda i: (i, 0)),
            pl.BlockSpec((T_E, 2), lambda i: (i, 0)),
            pl.BlockSpec((D, D * D), lambda i: (0, 0)),
            pl.BlockSpec((D_EH, D * D), lambda i: (0, 0)),
            pl.BlockSpec((1, D * D), lambda i: (0, 0)),
        ],
        out_specs=pl.BlockSpec((T_E, DP), lambda i: (i, 0)),
        out_shape=jax.ShapeDtypeStruct((E_PAD, DP), jnp.float32),
    )(x_src, h_p, par_p, rep, en2_w, en2_b.reshape(1, D * D))


def _gather_body(table_hbm, src_hbm, out_hbm, idx_v, rows_a, rows_b, rows_c,
                 gsem, wsem):
    wid = lax.axis_index("s") * 2 + lax.axis_index("c")
    pltpu.sync_copy(src_hbm.at[wid], idx_v)
    bufs = (rows_a, rows_b, rows_c)
    nq = K_CH // 2
    gathers = []
    writes = []

    def fire(q):
        b = bufs[q % 3]
        gathers.append((
            pltpu.async_copy(table_hbm.at[idx_v.at[2 * q]],
                             b.at[pl.ds(0, 128)], gsem),
            pltpu.async_copy(table_hbm.at[idx_v.at[2 * q + 1]],
                             b.at[pl.ds(128, 128)], gsem)))

    fire(0)
    fire(1)
    for q in range(nq):
        g0, g1 = gathers[q]
        g0.wait()
        g1.wait()
        writes.append(pltpu.async_copy(
            bufs[q % 3], out_hbm.at[pl.ds(wid * E_W + q * 256, 256)], wsem))
        if q + 2 < nq:
            if q >= 1:
                writes[q - 1].wait()
            fire(q + 2)
    writes[-3].wait()
    writes[-2].wait()
    writes[-1].wait()


@jax.jit
def _gather_call(table, src_p):
    mesh = plsc.VectorSubcoreMesh(core_axis_name="c", subcore_axis_name="s")
    return pl.kernel(
        _gather_body,
        out_type=jax.ShapeDtypeStruct((E_PAD, DP), jnp.float32),
        mesh=mesh,
        scratch_types=[
            pltpu.VMEM((K_CH, 128), jnp.int32),
            pltpu.VMEM((256, DP), jnp.float32),
            pltpu.VMEM((256, DP), jnp.float32),
            pltpu.VMEM((256, DP), jnp.float32),
            pltpu.SemaphoreType.DMA,
            pltpu.SemaphoreType.DMA,
        ],
    )(table, src_p)


def _scatter_body(msgs_hbm, dst_hbm, z_hbm, out_hbm, idx_v, rows_v, acc_sh, sem):
    c = lax.axis_index("c")
    s = lax.axis_index("s")
    wid = s * 2 + c
    pltpu.sync_copy(z_hbm, rows_v.at[pl.ds(0, ZROWS)])
    pltpu.sync_copy(rows_v.at[pl.ds(0, ZROWS)], acc_sh.at[pl.ds(s * ZROWS, ZROWS)])
    pltpu.sync_copy(dst_hbm.at[wid], idx_v)
    plsc.subcore_barrier()
    for h in range(2):
        pltpu.sync_copy(msgs_hbm.at[pl.ds(wid * E_W + h * E_H, E_H)], rows_v)
        for j in range(K_H):
            pltpu.sync_copy(rows_v.at[pl.ds(j * 128, 128)],
                            acc_sh.at[idx_v.at[h * K_H + j]], add=True)
    plsc.subcore_barrier()
    pltpu.sync_copy(acc_sh.at[pl.ds(s * ZROWS, ZROWS)], rows_v.at[pl.ds(0, ZROWS)])
    pltpu.sync_copy(rows_v.at[pl.ds(0, ZROWS)],
                    out_hbm.at[pl.ds(c * N_ACC + s * ZROWS, ZROWS)])


@jax.jit
def _scatter_call(msgs, dst_p, zblk):
    mesh = plsc.VectorSubcoreMesh(core_axis_name="c", subcore_axis_name="s")
    return pl.kernel(
        _scatter_body,
        out_type=jax.ShapeDtypeStruct((2 * N_ACC, DP), jnp.float32),
        mesh=mesh,
        scratch_types=[
            pltpu.VMEM((K_CH, 128), jnp.int32),
            pltpu.VMEM((E_H, DP), jnp.float32),
            pltpu.VMEM_SHARED((N_ACC, DP), jnp.float32),
            pltpu.SemaphoreType.DMA,
        ],
    )(msgs, dst_p, zblk)


def _update_body(p0_ref, p1_ref, out_ref, init_ref, mw1_ref, mw2_ref, mb_ref,
                 cb_ref, o_ref, *, add_init):
    out = out_ref[:, :D]
    neigh = p0_ref[:N, :] + p1_ref[:N, :]
    m = jnp.maximum(neigh + out + cb_ref[...], 0.0)
    r = (jnp.dot(m, mw1_ref[...], preferred_element_type=jnp.float32)
         + jnp.dot(out, mw2_ref[...], preferred_element_type=jnp.float32)
         + mb_ref[...])
    if add_init:
        r = r + init_ref[...]
        o_ref[...] = r
    else:
        o_ref[...] = jnp.concatenate(
            [r, jnp.zeros((N, DP - D), jnp.float32)], axis=1)


@functools.partial(jax.jit, static_argnames=("add_init",))
def _update_call(p0, p1, out, init, mw1, mw2, msg_b, conv_bias, add_init):
    return pl.pallas_call(
        functools.partial(_update_body, add_init=add_init),
        out_shape=jax.ShapeDtypeStruct((N, D if add_init else DP), jnp.float32),
    )(p0, p1, out, init, mw1, mw2, msg_b.reshape(1, D), conv_bias.reshape(1, D))


def kernel(n_feat, e_feat, lin0_w, lin0_b, en1_w, en1_b, en2_w, en2_b,
           msg_w, msg_b, conv_bias, edge_index):
    src = edge_index[0]
    dst = edge_index[1]
    src_p = jnp.zeros((E_PAD,), jnp.int32).at[:E].set(src).reshape(NW, K_CH, 128)
    dst_full = jnp.full((E_PAD,), N, jnp.int32).at[:E].set(dst)
    dst_p = (dst_full // 2).reshape(NW, K_CH, 128)
    parity = (dst_full % 2).astype(jnp.float32)
    par_p = jnp.stack([1.0 - parity, parity], axis=1)
    ef_p = jnp.zeros((E_PAD, D_EDGE), jnp.float32).at[:E].set(e_feat)
    zblk = jnp.zeros((ZROWS, DP), jnp.float32)
    rep = (jnp.arange(D * D, dtype=jnp.int32)[None, :] // D
           == jnp.arange(D, dtype=jnp.int32)[:, None]).astype(jnp.float32)
    mw1 = msg_w[:D]
    mw2 = msg_w[D:]

    out = _lin0_call(n_feat, lin0_w, lin0_b)
    h_p = _hpre_call(ef_p, en1_w, en1_b)
    for s in range(STEPS):
        x_src = _gather_call(out, src_p)
        msgs = _msgs_call(x_src, h_p, par_p, rep, en2_w, en2_b)
        parts = _scatter_call(msgs, dst_p, zblk)
        p0 = parts[:N_ACC].reshape(2 * N_ACC, D)
        p1 = parts[N_ACC:].reshape(2 * N_ACC, D)
        out = _update_call(p0, p1, out, n_feat, mw1, mw2, msg_b, conv_bias,
                           add_init=(s == STEPS - 1))
    return out


# speedup vs baseline: 1.6368x; 1.0047x over previous
import functools

import jax
import jax.numpy as jnp
from jax import lax
from jax.experimental import pallas as pl
from jax.experimental.pallas import tpu as pltpu
from jax.experimental.pallas import tpu_sc as plsc

N = 10000
E = 40000
D = 64
DP = 128
D_EDGE = 16
D_EH = 128
STEPS = 6

NW = 32
E_W = 640
EH_PAD = NW * E_W
E_PAD = 2 * EH_PAD
K_CH = E_W // 128
E_H = 640
N_ACC = 5120
ZROWS = N_ACC // 16
T_E = 1024


def _lin0_body(x_ref, w_ref, b_ref, o_ref):
    r = jnp.maximum(
        jnp.dot(x_ref[...], w_ref[...], preferred_element_type=jnp.float32)
        + b_ref[...], 0.0)
    o_ref[...] = jnp.concatenate([r, jnp.zeros((N, DP - D), jnp.float32)], axis=1)


@jax.jit
def _lin0_call(n_feat, lin0_w, lin0_b):
    return pl.pallas_call(
        _lin0_body,
        out_shape=jax.ShapeDtypeStruct((N, DP), jnp.float32),
    )(n_feat, lin0_w, lin0_b.reshape(1, D))


def _hpre_body(ef_ref, w1_ref, b1_ref, o_ref):
    o_ref[...] = jnp.maximum(
        jnp.dot(ef_ref[...], w1_ref[...], preferred_element_type=jnp.float32)
        + b1_ref[...], 0.0)


@jax.jit
def _hpre_call(ef_p, en1_w, en1_b):
    return pl.pallas_call(
        _hpre_body,
        grid=(8,),
        in_specs=[
            pl.BlockSpec((E_PAD // 8, D_EDGE), lambda i: (i, 0)),
            pl.BlockSpec((D_EDGE, D_EH), lambda i: (0, 0)),
            pl.BlockSpec((1, D_EH), lambda i: (0, 0)),
        ],
        out_specs=pl.BlockSpec((E_PAD // 8, D_EH), lambda i: (i, 0)),
        out_shape=jax.ShapeDtypeStruct((E_PAD, D_EH), jnp.float32),
    )(ef_p, en1_w, en1_b.reshape(1, D_EH))


def _msgs_body(x_ref, h_ref, par_ref, rep_ref, w2_ref, b2_ref, o_ref):
    w = jnp.dot(h_ref[...], w2_ref[...], preferred_element_type=jnp.float32) \
        + b2_ref[...]
    xrep = jnp.dot(x_ref[:, :D], rep_ref[...],
                   preferred_element_type=jnp.float32)
    s = w * xrep
    width = D * D // 2
    while width >= D:
        s = s[:, :width] + s[:, width:2 * width]
        width //= 2
    o_ref[...] = jnp.concatenate(
        [s * par_ref[:, 0:1], s * par_ref[:, 1:2]], axis=1)


@jax.jit
def _msgs_call(x_src, h_p, par_p, rep, en2_w, en2_b):
    return pl.pallas_call(
        _msgs_body,
        grid=(EH_PAD // T_E,),
        in_specs=[
            pl.BlockSpec((T_E, DP), lambda i: (i, 0)),
            pl.BlockSpec((T_E, D_EH), lambda i: (i, 0)),
            pl.BlockSpec((T_E, 2), lambda i: (i, 0)),
            pl.BlockSpec((D, D * D), lambda i: (0, 0)),
            pl.BlockSpec((D_EH, D * D), lambda i: (0, 0)),
            pl.BlockSpec((1, D * D), lambda i: (0, 0)),
        ],
        out_specs=pl.BlockSpec((T_E, DP), lambda i: (i, 0)),
        out_shape=jax.ShapeDtypeStruct((EH_PAD, DP), jnp.float32),
    )(x_src, h_p, par_p, rep, en2_w, en2_b.reshape(1, D * D))


def _gather_body(table_hbm, src_hbm, out_hbm, idx_v, rows_a, rows_b, rows_c,
                 rows_d, gsem, wsem):
    wid = lax.axis_index("s") * 2 + lax.axis_index("c")
    pltpu.sync_copy(src_hbm.at[wid], idx_v)
    bufs = (rows_a, rows_b, rows_c, rows_d)
    gathers = []
    writes = []

    def fire(q):
        gathers.append(pltpu.async_copy(table_hbm.at[idx_v.at[q]],
                                        bufs[q % 4], gsem))

    fire(0)
    fire(1)
    fire(2)
    for q in range(K_CH):
        gathers[q].wait()
        writes.append(pltpu.async_copy(
            bufs[q % 4], out_hbm.at[pl.ds(wid * E_W + q * 128, 128)], wsem))
        if q + 3 < K_CH:
            if q >= 1:
                writes[q - 1].wait()
            fire(q + 3)
    for w in writes[-4:]:
        w.wait()


@jax.jit
def _gather_call(table, src_p):
    mesh = plsc.VectorSubcoreMesh(core_axis_name="c", subcore_axis_name="s")
    return pl.kernel(
        _gather_body,
        out_type=jax.ShapeDtypeStruct((EH_PAD, DP), jnp.float32),
        mesh=mesh,
        scratch_types=[
            pltpu.VMEM((K_CH, 128), jnp.int32),
            pltpu.VMEM((128, DP), jnp.float32),
            pltpu.VMEM((128, DP), jnp.float32),
            pltpu.VMEM((128, DP), jnp.float32),
            pltpu.VMEM((128, DP), jnp.float32),
            pltpu.SemaphoreType.DMA,
            pltpu.SemaphoreType.DMA,
        ],
    )(table, src_p)


def _scatter_body(msgs_hbm, dst_hbm, z_hbm, out_hbm, idx_v, rows_v, acc_sh, sem):
    c = lax.axis_index("c")
    s = lax.axis_index("s")
    wid = s * 2 + c
    pltpu.sync_copy(z_hbm, rows_v.at[pl.ds(0, ZROWS)])
    pltpu.sync_copy(rows_v.at[pl.ds(0, ZROWS)], acc_sh.at[pl.ds(s * ZROWS, ZROWS)])
    pltpu.sync_copy(dst_hbm.at[wid], idx_v)
    plsc.subcore_barrier()
    pltpu.sync_copy(msgs_hbm.at[pl.ds(wid * E_W, E_W)], rows_v.at[pl.ds(0, E_W)])
    for j in range(K_CH):
        pltpu.sync_copy(rows_v.at[pl.ds(j * 128, 128)],
                        acc_sh.at[idx_v.at[j]], add=True)
    plsc.subcore_barrier()
    pltpu.sync_copy(acc_sh.at[pl.ds(s * ZROWS, ZROWS)], rows_v.at[pl.ds(0, ZROWS)])
    pltpu.sync_copy(rows_v.at[pl.ds(0, ZROWS)],
                    out_hbm.at[pl.ds(c * N_ACC + s * ZROWS, ZROWS)])


@jax.jit
def _scatter_call(msgs, dst_p, zblk):
    mesh = plsc.VectorSubcoreMesh(core_axis_name="c", subcore_axis_name="s")
    return pl.kernel(
        _scatter_body,
        out_type=jax.ShapeDtypeStruct((2 * N_ACC, DP), jnp.float32),
        mesh=mesh,
        scratch_types=[
            pltpu.VMEM((K_CH, 128), jnp.int32),
            pltpu.VMEM((E_H, DP), jnp.float32),
            pltpu.VMEM_SHARED((N_ACC, DP), jnp.float32),
            pltpu.SemaphoreType.DMA,
        ],
    )(msgs, dst_p, zblk)


def _update_body(p0_ref, p1_ref, p2_ref, p3_ref, out_ref, init_ref, mw1_ref,
                 mw2_ref, mb_ref, cb_ref, o_ref, *, add_init):
    out = out_ref[:, :D]
    neigh = ((p0_ref[:N, :] + p1_ref[:N, :])
             + (p2_ref[:N, :] + p3_ref[:N, :]))
    m = jnp.maximum(neigh + out + cb_ref[...], 0.0)
    r = (jnp.dot(m, mw1_ref[...], preferred_element_type=jnp.float32)
         + jnp.dot(out, mw2_ref[...], preferred_element_type=jnp.float32)
         + mb_ref[...])
    if add_init:
        r = r + init_ref[...]
        o_ref[...] = r
    else:
        o_ref[...] = jnp.concatenate(
            [r, jnp.zeros((N, DP - D), jnp.float32)], axis=1)


@functools.partial(jax.jit, static_argnames=("add_init",))
def _update_call(p0, p1, p2, p3, out, init, mw1, mw2, msg_b, conv_bias,
                 add_init):
    return pl.pallas_call(
        functools.partial(_update_body, add_init=add_init),
        out_shape=jax.ShapeDtypeStruct((N, D if add_init else DP), jnp.float32),
    )(p0, p1, p2, p3, out, init, mw1, mw2, msg_b.reshape(1, D),
      conv_bias.reshape(1, D))


def kernel(n_feat, e_feat, lin0_w, lin0_b, en1_w, en1_b, en2_w, en2_b,
           msg_w, msg_b, conv_bias, edge_index):
    src = edge_index[0]
    dst = edge_index[1]
    src_full = jnp.zeros((E_PAD,), jnp.int32).at[:E].set(src)
    src_h = [src_full[h * EH_PAD:(h + 1) * EH_PAD].reshape(NW, K_CH, 128)
             for h in range(2)]
    dst_full = jnp.full((E_PAD,), N, jnp.int32).at[:E].set(dst)
    dst_h = [(dst_full[h * EH_PAD:(h + 1) * EH_PAD] // 2).reshape(NW, K_CH, 128)
             for h in range(2)]
    parity = (dst_full % 2).astype(jnp.float32)
    par_p = jnp.stack([1.0 - parity, parity], axis=1)
    par_h = [par_p[h * EH_PAD:(h + 1) * EH_PAD] for h in range(2)]
    ef_p = jnp.zeros((E_PAD, D_EDGE), jnp.float32).at[:E].set(e_feat)
    zblk = jnp.zeros((ZROWS, DP), jnp.float32)
    rep = (jnp.arange(D * D, dtype=jnp.int32)[None, :] // D
           == jnp.arange(D, dtype=jnp.int32)[:, None]).astype(jnp.float32)
    mw1 = msg_w[:D]
    mw2 = msg_w[D:]

    out = _lin0_call(n_feat, lin0_w, lin0_b)
    h_p = _hpre_call(ef_p, en1_w, en1_b)
    h_h = [h_p[h * EH_PAD:(h + 1) * EH_PAD] for h in range(2)]
    for s in range(STEPS):
        xA = _gather_call(out, src_h[0])
        xB = _gather_call(out, src_h[1])
        mA = _msgs_call(xA, h_h[0], par_h[0], rep, en2_w, en2_b)
        mB = _msgs_call(xB, h_h[1], par_h[1], rep, en2_w, en2_b)
        pa = _scatter_call(mA, dst_h[0], zblk)
        pb = _scatter_call(mB, dst_h[1], zblk)
        p0 = pa[:N_ACC].reshape(2 * N_ACC, D)
        p1 = pa[N_ACC:].reshape(2 * N_ACC, D)
        p2 = pb[:N_ACC].reshape(2 * N_ACC, D)
        p3 = pb[N_ACC:].reshape(2 * N_ACC, D)
        out = _update_call(p0, p1, p2, p3, out, n_feat, mw1, mw2, msg_b,
                           conv_bias, add_init=(s == STEPS - 1))
    return out
